# baseline (speedup 1.0000x reference)
import functools

import jax
import jax.numpy as jnp
from jax.experimental import pallas as pl
from jax.experimental.pallas import tpu as pltpu

S = 1024
BLK = 128
NB = S // BLK
NSTEP = 3 * NB


def _mv(v, W):
    return jax.lax.dot_general(
        v, W, (((1,), (1,)), ((), ())), preferred_element_type=jnp.float32
    )


def _fused_kernel(x_ref, h_ref, bih_ref, bhh_ref, bout_ref,
                  wih_ref, whh_ref, wo_ref, out_ref, hnew_ref,
                  gi_ref, gh_ref, lg_ref):
    j = pl.program_id(0)
    x = x_ref[...]
    h = h_ref[...]

    gi_ref[pl.ds(j, 1), :] = _mv(x, wih_ref[...])
    gh_ref[pl.ds(j, 1), :] = _mv(h, whh_ref[...])

    @pl.when(j % 3 == 0)
    def _():
        k = j // 3
        wo = wo_ref[...]
        lx = _mv(x, wo[:, :S])
        lh = _mv(h, wo[:, S:2 * S] + wo[:, 2 * S:3 * S])
        lg_ref[pl.ds(k, 1), :] = lx + lh + bout_ref[pl.ds(k, 1), :]

    @pl.when(j == NSTEP - 1)
    def _():
        gi = gi_ref[...] + bih_ref[...]
        gh = gh_ref[...] + bhh_ref[...]
        r = jax.nn.sigmoid(gi[:NB] + gh[:NB])
        z = jax.nn.sigmoid(gi[NB:2 * NB] + gh[NB:2 * NB])
        n = jnp.tanh(gi[2 * NB:] + r * gh[2 * NB:])
        hprev = h.reshape(NB, BLK)
        hnew_ref[...] = ((1.0 - z) * n + z * hprev).reshape(1, 1, S)

        logits = lg_ref[...]
        m = jnp.max(logits)
        lse = m + jnp.log(jnp.sum(jnp.exp(logits - m)))
        out_ref[...] = (logits - lse).reshape(1, S)


@functools.partial(jax.jit, static_argnames=())
def _run(feature, initial_h, W_ih, W_hh, b_ih, b_hh, W_out, b_out):
    x2 = feature.reshape(1, S)
    h2 = initial_h.reshape(1, S)
    bih = b_ih.reshape(3 * NB, BLK)
    bhh = b_hh.reshape(3 * NB, BLK)
    bout = b_out.reshape(NB, BLK)

    full = lambda shape: pl.BlockSpec(shape, lambda j: tuple(0 for _ in shape))

    out, h_new = pl.pallas_call(
        _fused_kernel,
        grid=(NSTEP,),
        in_specs=[
            full((1, S)),
            full((1, S)),
            full((3 * NB, BLK)),
            full((3 * NB, BLK)),
            full((NB, BLK)),
            pl.BlockSpec((BLK, S), lambda j: (j, 0)),
            pl.BlockSpec((BLK, S), lambda j: (j, 0)),
            pl.BlockSpec((BLK, 3 * S + 1), lambda j: (j // 3, 0)),
        ],
        out_specs=[
            pl.BlockSpec((1, S), lambda j: (0, 0)),
            pl.BlockSpec((1, 1, S), lambda j: (0, 0, 0)),
        ],
        out_shape=[
            jax.ShapeDtypeStruct((1, S), jnp.float32),
            jax.ShapeDtypeStruct((1, 1, S), jnp.float32),
        ],
        scratch_shapes=[
            pltpu.VMEM((3 * NB, BLK), jnp.float32),
            pltpu.VMEM((3 * NB, BLK), jnp.float32),
            pltpu.VMEM((NB, BLK), jnp.float32),
        ],
        compiler_params=pltpu.CompilerParams(
            dimension_semantics=("arbitrary",),
        ),
    )(x2, h2, bih, bhh, bout, W_ih, W_hh, W_out)
    return out, h_new


def kernel(feature, time, initial_h, W_ih, W_hh, b_ih, b_hh, W_out, b_out):
    del time
    return _run(feature, initial_h, W_ih, W_hh, b_ih, b_hh, W_out, b_out)


# speedup vs baseline: 1.3217x; 1.3217x over previous
import functools

import jax
import jax.numpy as jnp
from jax.experimental import pallas as pl
from jax.experimental.pallas import tpu as pltpu

S = 1024
NSTEP = 8
GR = 3 * S // NSTEP
OR = S // NSTEP
NB = S // 128


def _mv(v, W):
    return jax.lax.dot_general(
        v, W, (((1,), (1,)), ((), ())), preferred_element_type=jnp.float32
    )


def _fused_kernel(x_ref, h_ref, bih_ref, bhh_ref, bout_ref,
                  wih_ref, whh_ref, wo_ref, out_ref, hnew_ref,
                  gi_ref, gh_ref, lg_ref):
    j = pl.program_id(0)
    x = x_ref[...]
    h = h_ref[...]

    nsub = GR // 128
    gi_ref[pl.ds(nsub * j, nsub), :] = _mv(x, wih_ref[...]).reshape(nsub, 128)
    gh_ref[pl.ds(nsub * j, nsub), :] = _mv(h, whh_ref[...]).reshape(nsub, 128)

    wo = wo_ref[...]
    lx = _mv(x, wo[:, :S])
    lh = _mv(h, wo[:, S:2 * S] + wo[:, 2 * S:])
    lg_ref[pl.ds(j, 1), :] = lx + lh + bout_ref[pl.ds(j, 1), :]

    @pl.when(j == NSTEP - 1)
    def _():
        gi = gi_ref[...] + bih_ref[...]
        gh = gh_ref[...] + bhh_ref[...]
        r = jax.nn.sigmoid(gi[:NB] + gh[:NB])
        z = jax.nn.sigmoid(gi[NB:2 * NB] + gh[NB:2 * NB])
        n = jnp.tanh(gi[2 * NB:] + r * gh[2 * NB:])
        hprev = h.reshape(NB, 128)
        hnew_ref[...] = ((1.0 - z) * n + z * hprev).reshape(1, 1, S)

        logits = lg_ref[...]
        m = jnp.max(logits)
        lse = m + jnp.log(jnp.sum(jnp.exp(logits - m)))
        out_ref[...] = (logits - lse).reshape(1, S)


@functools.partial(jax.jit, static_argnames=())
def _run(feature, initial_h, W_ih, W_hh, b_ih, b_hh, W_out, b_out):
    x2 = feature.reshape(1, S)
    h2 = initial_h.reshape(1, S)
    bih = b_ih.reshape(3 * NB, 128)
    bhh = b_hh.reshape(3 * NB, 128)
    bout = b_out.reshape(NB, 128)

    full = lambda shape: pl.BlockSpec(shape, lambda j: tuple(0 for _ in shape))

    out, h_new = pl.pallas_call(
        _fused_kernel,
        grid=(NSTEP,),
        in_specs=[
            full((1, S)),
            full((1, S)),
            full((3 * NB, 128)),
            full((3 * NB, 128)),
            full((NB, 128)),
            pl.BlockSpec((GR, S), lambda j: (j, 0)),
            pl.BlockSpec((GR, S), lambda j: (j, 0)),
            pl.BlockSpec((OR, 3 * S), lambda j: (j, 0)),
        ],
        out_specs=[
            pl.BlockSpec((1, S), lambda j: (0, 0)),
            pl.BlockSpec((1, 1, S), lambda j: (0, 0, 0)),
        ],
        out_shape=[
            jax.ShapeDtypeStruct((1, S), jnp.float32),
            jax.ShapeDtypeStruct((1, 1, S), jnp.float32),
        ],
        scratch_shapes=[
            pltpu.VMEM((3 * NB, 128), jnp.float32),
            pltpu.VMEM((3 * NB, 128), jnp.float32),
            pltpu.VMEM((NB, 128), jnp.float32),
        ],
        compiler_params=pltpu.CompilerParams(
            dimension_semantics=("arbitrary",),
        ),
    )(x2, h2, bih, bhh, bout, W_ih, W_hh, W_out)
    return out, h_new


def kernel(feature, time, initial_h, W_ih, W_hh, b_ih, b_hh, W_out, b_out):
    del time
    return _run(feature, initial_h, W_ih, W_hh, b_ih, b_hh, W_out, b_out)


# speedup vs baseline: 1.3240x; 1.0018x over previous
import functools

import jax
import jax.numpy as jnp
from jax.experimental import pallas as pl
from jax.experimental.pallas import tpu as pltpu

S = 1024
NSTEP = 4
GR = 3 * S // NSTEP
OR = S // NSTEP
NB = S // 128


def _mv(v, W):
    return jax.lax.dot_general(
        v, W, (((1,), (1,)), ((), ())), preferred_element_type=jnp.float32
    )


def _fused_kernel(x_ref, h_ref, bih_ref, bhh_ref, bout_ref,
                  wih_ref, whh_ref, wo_ref, out_ref, hnew_ref,
                  gi_ref, gh_ref, lg_ref):
    j = pl.program_id(0)
    x = x_ref[...]
    h = h_ref[...]

    nsub = GR // 128
    gi_ref[pl.ds(nsub * j, nsub), :] = _mv(x, wih_ref[...]).reshape(nsub, 128)
    gh_ref[pl.ds(nsub * j, nsub), :] = _mv(h, whh_ref[...]).reshape(nsub, 128)

    wo = wo_ref[...]
    lx = _mv(x, wo[:, :S])
    lh = _mv(h, wo[:, S:2 * S] + wo[:, 2 * S:])
    osub = OR // 128
    lg_ref[pl.ds(osub * j, osub), :] = (
        (lx + lh).reshape(osub, 128) + bout_ref[pl.ds(osub * j, osub), :])

    @pl.when(j == NSTEP - 1)
    def _():
        gi = gi_ref[...] + bih_ref[...]
        gh = gh_ref[...] + bhh_ref[...]
        r = jax.nn.sigmoid(gi[:NB] + gh[:NB])
        z = jax.nn.sigmoid(gi[NB:2 * NB] + gh[NB:2 * NB])
        n = jnp.tanh(gi[2 * NB:] + r * gh[2 * NB:])
        hprev = h.reshape(NB, 128)
        hnew_ref[...] = ((1.0 - z) * n + z * hprev).reshape(1, 1, S)

        logits = lg_ref[...]
        m = jnp.max(logits)
        lse = m + jnp.log(jnp.sum(jnp.exp(logits - m)))
        out_ref[...] = (logits - lse).reshape(1, S)


@functools.partial(jax.jit, static_argnames=())
def _run(feature, initial_h, W_ih, W_hh, b_ih, b_hh, W_out, b_out):
    x2 = feature.reshape(1, S)
    h2 = initial_h.reshape(1, S)
    bih = b_ih.reshape(3 * NB, 128)
    bhh = b_hh.reshape(3 * NB, 128)
    bout = b_out.reshape(NB, 128)

    full = lambda shape: pl.BlockSpec(shape, lambda j: tuple(0 for _ in shape))

    out, h_new = pl.pallas_call(
        _fused_kernel,
        grid=(NSTEP,),
        in_specs=[
            full((1, S)),
            full((1, S)),
            full((3 * NB, 128)),
            full((3 * NB, 128)),
            full((NB, 128)),
            pl.BlockSpec((GR, S), lambda j: (j, 0)),
            pl.BlockSpec((GR, S), lambda j: (j, 0)),
            pl.BlockSpec((OR, 3 * S), lambda j: (j, 0)),
        ],
        out_specs=[
            pl.BlockSpec((1, S), lambda j: (0, 0)),
            pl.BlockSpec((1, 1, S), lambda j: (0, 0, 0)),
        ],
        out_shape=[
            jax.ShapeDtypeStruct((1, S), jnp.float32),
            jax.ShapeDtypeStruct((1, 1, S), jnp.float32),
        ],
        scratch_shapes=[
            pltpu.VMEM((3 * NB, 128), jnp.float32),
            pltpu.VMEM((3 * NB, 128), jnp.float32),
            pltpu.VMEM((NB, 128), jnp.float32),
        ],
        compiler_params=pltpu.CompilerParams(
            dimension_semantics=("arbitrary",),
        ),
    )(x2, h2, bih, bhh, bout, W_ih, W_hh, W_out)
    return out, h_new


def kernel(feature, time, initial_h, W_ih, W_hh, b_ih, b_hh, W_out, b_out):
    del time
    return _run(feature, initial_h, W_ih, W_hh, b_ih, b_hh, W_out, b_out)


# speedup vs baseline: 1.3353x; 1.0085x over previous
import functools

import jax
import jax.numpy as jnp
from jax.experimental import pallas as pl
from jax.experimental.pallas import tpu as pltpu

S = 1024
H = 512


def _mv(v, W):
    return jax.lax.dot_general(
        v, W, (((1,), (1,)), ((), ())), preferred_element_type=jnp.float32
    )


def _kernel_body(x_ref, h_ref, bih_ref, bhh_ref, bout_ref,
                 wih_hbm, whh_hbm, wo_hbm, out_ref, hnew_ref,
                 wih_v, whh_v, wo_v, sem):
    c_ih = pltpu.make_async_copy(wih_hbm, wih_v, sem.at[0])
    c_hh = pltpu.make_async_copy(whh_hbm, whh_v, sem.at[1])
    c_oa = pltpu.make_async_copy(wo_hbm.at[0:H, 0:3 * S], wo_v.at[0:H], sem.at[2])
    c_ob = pltpu.make_async_copy(wo_hbm.at[H:2 * H, 0:3 * S], wo_v.at[H:2 * H], sem.at[3])
    c_ih.start()
    c_hh.start()
    c_oa.start()
    c_ob.start()

    x = x_ref[...]
    h = h_ref[...]

    c_ih.wait()
    gi = _mv(x, wih_v[...]) + bih_ref[...]
    c_hh.wait()
    gh = _mv(h, whh_v[...]) + bhh_ref[...]

    r = jax.nn.sigmoid(gi[:, :S] + gh[:, :S])
    z = jax.nn.sigmoid(gi[:, S:2 * S] + gh[:, S:2 * S])
    n = jnp.tanh(gi[:, 2 * S:] + r * gh[:, 2 * S:])
    hnew_ref[...] = ((1.0 - z) * n + z * h).reshape(1, 1, S)

    c_oa.wait()
    wa = wo_v[0:H, :]
    la = _mv(x, wa[:, :S]) + _mv(h, wa[:, S:2 * S]) + _mv(h, wa[:, 2 * S:])
    c_ob.wait()
    wb = wo_v[H:2 * H, :]
    lb = _mv(x, wb[:, :S]) + _mv(h, wb[:, S:2 * S]) + _mv(h, wb[:, 2 * S:])
    logits = jnp.concatenate([la, lb], axis=1) + bout_ref[...]

    m = jnp.max(logits)
    lse = m + jnp.log(jnp.sum(jnp.exp(logits - m)))
    out_ref[...] = logits - lse


@functools.partial(jax.jit, static_argnames=())
def _run(feature, initial_h, W_ih, W_hh, b_ih, b_hh, W_out, b_out):
    x2 = feature.reshape(1, S)
    h2 = initial_h.reshape(1, S)
    bih = b_ih.reshape(1, 3 * S)
    bhh = b_hh.reshape(1, 3 * S)
    bout = b_out.reshape(1, S)

    vm = lambda: pl.BlockSpec(memory_space=pltpu.VMEM)
    anym = lambda: pl.BlockSpec(memory_space=pltpu.HBM)

    out, h_new = pl.pallas_call(
        _kernel_body,
        in_specs=[vm(), vm(), vm(), vm(), vm(), anym(), anym(), anym()],
        out_specs=[vm(), vm()],
        out_shape=[
            jax.ShapeDtypeStruct((1, S), jnp.float32),
            jax.ShapeDtypeStruct((1, 1, S), jnp.float32),
        ],
        scratch_shapes=[
            pltpu.VMEM((3 * S, S), jnp.float32),
            pltpu.VMEM((3 * S, S), jnp.float32),
            pltpu.VMEM((S, 3 * S), jnp.float32),
            pltpu.SemaphoreType.DMA((4,)),
        ],
    )(x2, h2, bih, bhh, bout, W_ih, W_hh, W_out)
    return out, h_new


def kernel(feature, time, initial_h, W_ih, W_hh, b_ih, b_hh, W_out, b_out):
    del time
    return _run(feature, initial_h, W_ih, W_hh, b_ih, b_hh, W_out, b_out)
